# Initial kernel scaffold; baseline (speedup 1.0000x reference)
import jax
import jax.numpy as jnp
from jax.experimental import pallas as pl


def kernel(x, edge_index, W0_1, b0_1, W0_2, b0_2, W1_1, b1_1, W1_2, b1_2, W2_1, b2_1, W2_2, b2_2):
    raise NotImplementedError("write your pallas kernel here")


# speedup vs baseline: 5.0961x; 5.0961x over previous
import functools

import jax
import jax.numpy as jnp
from jax import lax
from jax.experimental import pallas as pl
from jax.experimental.pallas import tpu as pltpu
from jax.experimental.pallas import tpu_sc as plsc

N = 10000
E = 320000
F = 128

NC = 2
NS = 16
NW = NC * NS
E_PER_W = E // NW
CHUNK = 80
N_CHUNKS = E_PER_W // CHUNK
N_PAD = 10240
ROWS_PER_TILE = N_PAD // NS
ZROWS = 128


def _agg_body(x_hbm, ei_hbm, out_hbm, src_v, dst_v, rows_v, zbuf, agg_sh, sem):
  c = lax.axis_index("c")
  s = lax.axis_index("s")
  wid = s * NC + c

  def _zrow(i, _):
    def _zcol(j, _):
      zbuf[i, pl.ds(j * 16, 16)] = jnp.zeros((16,), jnp.float32)
      return 0
    return lax.fori_loop(0, F // 16, _zcol, 0)
  lax.fori_loop(0, ZROWS, _zrow, 0)
  for r in range(ROWS_PER_TILE // ZROWS):
    pltpu.sync_copy(zbuf, agg_sh.at[pl.ds(s * ROWS_PER_TILE + r * ZROWS, ZROWS)])
  plsc.subcore_barrier()

  def _chunk(i, _):
    base = wid * E_PER_W + i * CHUNK
    pltpu.sync_copy(ei_hbm.at[pl.ds(base, CHUNK)], src_v)
    pltpu.sync_copy(ei_hbm.at[pl.ds(E + base, CHUNK)], dst_v)
    pltpu.async_copy(x_hbm.at[src_v], rows_v, sem).wait()
    pltpu.sync_copy(rows_v, agg_sh.at[dst_v], add=True)
    return 0
  lax.fori_loop(0, N_CHUNKS, _chunk, 0)
  plsc.subcore_barrier()

  pltpu.sync_copy(agg_sh.at[pl.ds(s * ROWS_PER_TILE, ROWS_PER_TILE)],
                  out_hbm.at[c, pl.ds(s * ROWS_PER_TILE, ROWS_PER_TILE)])


_agg = functools.partial(
    pl.kernel,
    out_type=jax.ShapeDtypeStruct((NC, N_PAD, F), jnp.float32),
    mesh=plsc.VectorSubcoreMesh(core_axis_name="c", subcore_axis_name="s"),
    scratch_types=[
        pltpu.VMEM((CHUNK,), jnp.int32),
        pltpu.VMEM((CHUNK,), jnp.int32),
        pltpu.VMEM((CHUNK, F), jnp.float32),
        pltpu.VMEM((ZROWS, F), jnp.float32),
        pltpu.VMEM_SHARED((N_PAD, F), jnp.float32),
        pltpu.SemaphoreType.DMA,
    ],
)(_agg_body)


def _mlp_body(last, a0_ref, a1_ref, x_ref, w1_ref, b1_ref, w2_ref, b2_ref, o_ref):
  h = x_ref[...] + a0_ref[0] + a1_ref[0]
  t = jnp.maximum(jnp.dot(h, w1_ref[...], preferred_element_type=jnp.float32)
                  + b1_ref[...], 0.0)
  o = jnp.dot(t, w2_ref[...], preferred_element_type=jnp.float32) + b2_ref[...]
  if last:
    m = jnp.max(o, axis=-1, keepdims=True)
    lse = m + jnp.log(jnp.sum(jnp.exp(o - m), axis=-1, keepdims=True))
    o_ref[...] = o - lse
  else:
    o_ref[...] = jnp.maximum(o, 0.0) + x_ref[...]


def _mlp(agg, x, W1, b1, W2, b2, last):
  blk = 1000
  grid = N // blk
  out_ch = W2.shape[1]
  return pl.pallas_call(
      functools.partial(_mlp_body, last),
      grid=(grid,),
      in_specs=[
          pl.BlockSpec((1, blk, F), lambda i: (0, i, 0)),
          pl.BlockSpec((1, blk, F), lambda i: (1, i, 0)),
          pl.BlockSpec((blk, F), lambda i: (i, 0)),
          pl.BlockSpec((F, W1.shape[1]), lambda i: (0, 0)),
          pl.BlockSpec((W1.shape[1],), lambda i: (0,)),
          pl.BlockSpec((W1.shape[1], out_ch), lambda i: (0, 0)),
          pl.BlockSpec((out_ch,), lambda i: (0,)),
      ],
      out_specs=pl.BlockSpec((blk, out_ch), lambda i: (i, 0)),
      out_shape=jax.ShapeDtypeStruct((N, out_ch), jnp.float32),
  )(agg, agg, x, W1, b1, W2, b2)


def kernel(x, edge_index, W0_1, b0_1, W0_2, b0_2, W1_1, b1_1, W1_2, b1_2,
           W2_1, b2_1, W2_2, b2_2):
  edge_index = edge_index.reshape(-1)
  agg = _agg(x, edge_index)
  x = _mlp(agg, x, W0_1, b0_1, W0_2, b0_2, last=False)
  agg = _agg(x, edge_index)
  x = _mlp(agg, x, W1_1, b1_1, W1_2, b1_2, last=False)
  agg = _agg(x, edge_index)
  return _mlp(agg, x, W2_1, b2_1, W2_2, b2_2, last=True)
